# Initial kernel scaffold; baseline (speedup 1.0000x reference)
import jax
import jax.numpy as jnp
from jax.experimental import pallas as pl


def kernel(x, edge_index, edge_attr, Wx1, bx1, We1, be1, Wo1, bo1, Wx2, bx2, We2, be2, Wo2, bo2, Wx3, bx3, We3, be3, Wo3, bo3, Wl1, bl1, Wl2, bl2):
    raise NotImplementedError("write your pallas kernel here")


# speedup vs baseline: 14.8458x; 14.8458x over previous
import functools

import jax
import jax.numpy as jnp
from jax import lax
from jax.experimental import pallas as pl
from jax.experimental.pallas import tpu as pltpu
from jax.experimental.pallas import tpu_sc as plsc

F32 = jnp.float32

_N = 99996
_E = 6399744
_NPAD = 100352
_NW = 32
_W = 1024
_KW = 196
_PW = _W * _KW
_EPAD = _NW * _PW
_CH = 128
_NC = _W // _CH
_RPT = 6272


def _sc_conv(table, src2, dst2, a0, a1, gate):
    mesh = plsc.VectorSubcoreMesh(core_axis_name="c", subcore_axis_name="s")

    @functools.partial(
        pl.kernel,
        mesh=mesh,
        compiler_params=pltpu.CompilerParams(use_tc_tiling_on_sc=False),
        out_type=jax.ShapeDtypeStruct((2, _NPAD, 16), F32),
        scratch_types=[
            pltpu.VMEM((_NC, _CH), jnp.int32),
            pltpu.VMEM((_NC, _CH), jnp.int32),
            pltpu.VMEM((_W,), F32),
            pltpu.VMEM((_W,), F32),
            pltpu.VMEM((_W, 16), F32),
            pltpu.VMEM((4, 16), F32),
            pltpu.VMEM_SHARED((_NPAD, 16), F32),
            pltpu.SemaphoreType.DMA,
            pltpu.SemaphoreType.DMA,
        ],
    )
    def conv(table_h, src_h, dst_h, a0_h, a1_h, gate_h, out_h,
             src_v, dst_v, a0_v, a1_v, rows_v, g_v, acc, sem_g, sem_s):
        c = lax.axis_index("c")
        s = lax.axis_index("s")
        w = c * 16 + s

        def zbody(i, _):
            rows_v[i, :] = jnp.zeros((16,), F32)
            return 0
        lax.fori_loop(0, _W, zbody, 0)
        for t in range(6):
            pltpu.sync_copy(rows_v, acc.at[pl.ds(s * _RPT + t * _W, _W), :])
        pltpu.sync_copy(rows_v.at[pl.ds(0, 128), :],
                        acc.at[pl.ds(s * _RPT + 6 * _W, 128), :])
        plsc.subcore_barrier()

        pltpu.sync_copy(gate_h, g_v)

        def window(k, _):
            row0 = w * (_PW // _CH) + k * _NC
            e0 = w * _PW + k * _W
            pltpu.sync_copy(src_h.at[pl.ds(row0, _NC), :], src_v)
            pltpu.sync_copy(dst_h.at[pl.ds(row0, _NC), :], dst_v)
            pltpu.sync_copy(a0_h.at[pl.ds(e0, _W)], a0_v)
            pltpu.sync_copy(a1_h.at[pl.ds(e0, _W)], a1_v)

            hs = []
            for j in range(_NC):
                hs.append(pltpu.async_copy(
                    table_h.at[src_v.at[j]],
                    rows_v.at[pl.ds(j * _CH, _CH), :], sem_g))
            for h in hs:
                h.wait()

            u0 = g_v[0, :]
            u1 = g_v[1, :]
            cc = g_v[2, :]
            def gbody(g, _):
                av0 = a0_v[pl.ds(g * 16, 16)]
                av1 = a1_v[pl.ds(g * 16, 16)]
                for l in range(16):
                    i = g * 16 + l
                    gv = u0 * av0[l] + u1 * av1[l] + cc
                    rows_v[i, :] = rows_v[i, :] * gv
                return 0

            lax.fori_loop(0, _W // 16, gbody, 0)

            hs = []
            for j in range(_NC):
                hs.append(pltpu.async_copy(
                    rows_v.at[pl.ds(j * _CH, _CH), :],
                    acc.at[dst_v.at[j]], sem_s, add=True))
            for h in hs:
                h.wait()
            return 0

        lax.fori_loop(0, _KW, window, 0)
        plsc.subcore_barrier()
        pltpu.sync_copy(acc.at[pl.ds(s * _RPT, _RPT), :],
                        out_h.at[c, pl.ds(s * _RPT, _RPT), :])

    return conv(table, src2, dst2, a0, a1, gate)


def _prop1_tc(x_p, w_row, b_row):
    def body(x_ref, w_ref, b_ref, o_ref):
        o_ref[...] = x_ref[...] * w_ref[...] + b_ref[...]
    grid = (_NPAD // _RPT,)
    return pl.pallas_call(
        body,
        grid=grid,
        in_specs=[
            pl.BlockSpec((_RPT, 1), lambda i: (i, 0)),
            pl.BlockSpec((1, 16), lambda i: (0, 0)),
            pl.BlockSpec((1, 16), lambda i: (0, 0)),
        ],
        out_specs=pl.BlockSpec((_RPT, 16), lambda i: (i, 0)),
        out_shape=jax.ShapeDtypeStruct((_NPAD, 16), F32),
    )(x_p, w_row, b_row)


def _dense_tc(p, w1, b1, w2, b2):
    def body(p_ref, w1_ref, b1_ref, w2_ref, b2_ref, o_ref):
        agg = p_ref[0] + p_ref[1]
        h = jnp.maximum(
            jnp.dot(agg, w1_ref[...], preferred_element_type=F32,
                    precision=lax.Precision.HIGHEST) + b1_ref[...], 0.0)
        o_ref[...] = jnp.dot(h, w2_ref[...], preferred_element_type=F32,
                             precision=lax.Precision.HIGHEST) + b2_ref[...]
    grid = (_NPAD // _RPT,)
    return pl.pallas_call(
        body,
        grid=grid,
        in_specs=[
            pl.BlockSpec((2, _RPT, 16), lambda i: (0, i, 0)),
            pl.BlockSpec((16, 16), lambda i: (0, 0)),
            pl.BlockSpec((1, 16), lambda i: (0, 0)),
            pl.BlockSpec((16, 16), lambda i: (0, 0)),
            pl.BlockSpec((1, 16), lambda i: (0, 0)),
        ],
        out_specs=pl.BlockSpec((_RPT, 16), lambda i: (i, 0)),
        out_shape=jax.ShapeDtypeStruct((_NPAD, 16), F32),
    )(p, w1, b1, w2, b2)


def _dense3_tc(p, w1, b1):
    def body(p_ref, w1_ref, b1_ref, o_ref):
        agg = p_ref[0] + p_ref[1]
        o_ref[...] = jnp.maximum(
            jnp.dot(agg, w1_ref[...], preferred_element_type=F32,
                    precision=lax.Precision.HIGHEST) + b1_ref[...], 0.0)
    grid = (_NPAD // _RPT,)
    return pl.pallas_call(
        body,
        grid=grid,
        in_specs=[
            pl.BlockSpec((2, _RPT, 16), lambda i: (0, i, 0)),
            pl.BlockSpec((16, 16), lambda i: (0, 0)),
            pl.BlockSpec((1, 16), lambda i: (0, 0)),
        ],
        out_specs=pl.BlockSpec((_RPT, 16), lambda i: (i, 0)),
        out_shape=jax.ShapeDtypeStruct((_NPAD, 16), F32),
    )(p, w1, b1)


def _head_tc(h24, w1, b1, w2, b2):
    def body(x_ref, w1_ref, b1_ref, w2_ref, b2_ref, o_ref):
        h = jnp.maximum(
            jnp.dot(x_ref[...], w1_ref[...], preferred_element_type=F32,
                    precision=lax.Precision.HIGHEST) + b1_ref[...], 0.0)
        o = jnp.dot(h, w2_ref[...], preferred_element_type=F32,
                    precision=lax.Precision.HIGHEST) + b2_ref[...]
        m = jnp.max(o, axis=1, keepdims=True)
        lse = jnp.log(jnp.sum(jnp.exp(o - m), axis=1, keepdims=True)) + m
        o_ref[...] = o - lse
    rows = _N // 6
    return pl.pallas_call(
        body,
        out_shape=jax.ShapeDtypeStruct((rows, 4), F32),
    )(h24, w1, b1, w2, b2)


def _pad16(a, rows=16):
    out = jnp.zeros((rows, 16), F32)
    return out.at[:a.shape[0], :a.shape[1]].set(a)


def kernel(x, edge_index, edge_attr, Wx1, bx1, We1, be1, Wo1, bo1,
           Wx2, bx2, We2, be2, Wo2, bo2,
           Wx3, bx3, We3, be3, Wo3, bo3,
           Wl1, bl1, Wl2, bl2):
    src = edge_index[0]
    dst = edge_index[1]
    npad = _EPAD - _E
    fill_src = (jnp.arange(npad, dtype=jnp.int32) % _N)
    fill_dst = _N + (jnp.arange(npad, dtype=jnp.int32) % (_NPAD - _N))
    src_p = jnp.concatenate([src, fill_src]).reshape(_EPAD // _CH, _CH)
    dst_p = jnp.concatenate([dst, fill_dst]).reshape(_EPAD // _CH, _CH)
    zf = jnp.zeros((npad,), F32)
    a0 = jnp.concatenate([edge_attr[:, 0], zf])
    a1 = jnp.concatenate([edge_attr[:, 1], zf])

    def gate_mat(We, be):
        g = jnp.zeros((4, 16), F32)
        g = g.at[0, :9].set(We[:, 0])
        g = g.at[1, :9].set(We[:, 1])
        g = g.at[2, :9].set(be)
        return g

    ge1, ge2, ge3 = gate_mat(We1, be1), gate_mat(We2, be2), gate_mat(We3, be3)

    w1row = _pad16(Wx1.T, rows=1)
    b1row = _pad16(bx1[None, :], rows=1)
    Wo1e, bo1e = _pad16(Wo1.T), _pad16(bo1[None, :], rows=1)
    Wx2e, bx2e = _pad16(Wx2.T), _pad16(bx2[None, :], rows=1)
    Wo2e, bo2e = _pad16(Wo2.T), _pad16(bo2[None, :], rows=1)
    Wx3e, bx3e = _pad16(Wx3.T), _pad16(bx3[None, :], rows=1)
    Wo3e, bo3e = _pad16(Wo3.T), _pad16(bo3[None, :], rows=1)

    x_p = jnp.pad(x, ((0, _NPAD - _N), (0, 0)))

    t1 = _prop1_tc(x_p, w1row, b1row)
    p1 = _sc_conv(t1, src_p, dst_p, a0, a1, ge1)
    t2 = _dense_tc(p1, Wo1e, bo1e, Wx2e, bx2e)
    p2 = _sc_conv(t2, src_p, dst_p, a0, a1, ge2)
    t3 = _dense_tc(p2, Wo2e, bo2e, Wx3e, bx3e)
    p3 = _sc_conv(t3, src_p, dst_p, a0, a1, ge3)
    h3 = _dense3_tc(p3, Wo3e, bo3e)

    h24 = h3[:_N, :4].reshape(_N // 6, 24)
    return _head_tc(h24, Wl1.T, bl1[None, :], Wl2.T, bl2[None, :])


# speedup vs baseline: 16.7636x; 1.1292x over previous
import functools

import jax
import jax.numpy as jnp
from jax import lax
from jax.experimental import pallas as pl
from jax.experimental.pallas import tpu as pltpu
from jax.experimental.pallas import tpu_sc as plsc

F32 = jnp.float32

_N = 99996
_E = 6399744
_NPAD = 100352
_NW = 32
_W = 1024
_KW = 196
_PW = _W * _KW
_EPAD = _NW * _PW
_CH = 128
_NC = _W // _CH
_RPT = 6272


def _sc_conv(table, src2, dst2, a0, a1, gate):
    mesh = plsc.VectorSubcoreMesh(core_axis_name="c", subcore_axis_name="s")

    @functools.partial(
        pl.kernel,
        mesh=mesh,
        compiler_params=pltpu.CompilerParams(use_tc_tiling_on_sc=False),
        out_type=jax.ShapeDtypeStruct((2, _NPAD, 16), F32),
        scratch_types=[
            pltpu.VMEM((_NC, _CH), jnp.int32),
            pltpu.VMEM((_NC, _CH), jnp.int32),
            pltpu.VMEM((_W,), F32),
            pltpu.VMEM((_W,), F32),
            pltpu.VMEM((_W, 16), F32),
            pltpu.VMEM((4, 16), F32),
            pltpu.VMEM_SHARED((_NPAD, 16), F32),
            pltpu.SemaphoreType.DMA,
            pltpu.SemaphoreType.DMA,
        ],
    )
    def conv(table_h, src_h, dst_h, a0_h, a1_h, gate_h, out_h,
             src_v, dst_v, a0_v, a1_v, rows_v, g_v, acc, sem_g, sem_s):
        c = lax.axis_index("c")
        s = lax.axis_index("s")
        w = c * 16 + s

        def zbody(i, _):
            rows_v[i, :] = jnp.zeros((16,), F32)
            return 0
        lax.fori_loop(0, _W, zbody, 0)
        for t in range(6):
            pltpu.sync_copy(rows_v, acc.at[pl.ds(s * _RPT + t * _W, _W), :])
        pltpu.sync_copy(rows_v.at[pl.ds(0, 128), :],
                        acc.at[pl.ds(s * _RPT + 6 * _W, 128), :])
        plsc.subcore_barrier()

        pltpu.sync_copy(gate_h, g_v)

        def window(k, _):
            row0 = w * (_PW // _CH) + k * _NC
            e0 = w * _PW + k * _W
            pltpu.sync_copy(src_h.at[pl.ds(row0, _NC), :], src_v)
            pltpu.sync_copy(dst_h.at[pl.ds(row0, _NC), :], dst_v)
            pltpu.sync_copy(a0_h.at[pl.ds(e0, _W)], a0_v)
            pltpu.sync_copy(a1_h.at[pl.ds(e0, _W)], a1_v)

            hs = []
            for j in range(_NC):
                hs.append(pltpu.async_copy(
                    table_h.at[src_v.at[j]],
                    rows_v.at[pl.ds(j * _CH, _CH), :], sem_g))
            for h in hs:
                h.wait()

            u0 = g_v[0, :]
            u1 = g_v[1, :]
            cc = g_v[2, :]
            def gbody(g, _):
                av0 = a0_v[pl.ds(g * 16, 16)]
                av1 = a1_v[pl.ds(g * 16, 16)]
                for l in range(16):
                    i = g * 16 + l
                    gv = u0 * av0[l] + u1 * av1[l] + cc
                    rows_v[i, :] = rows_v[i, :] * gv
                return 0

            lax.fori_loop(0, _W // 16, gbody, 0)

            hs = []
            for j in range(_NC):
                hs.append(pltpu.async_copy(
                    rows_v.at[pl.ds(j * _CH, _CH), :],
                    acc.at[dst_v.at[j]], sem_s, add=True))
            for h in hs:
                h.wait()
            return 0

        lax.fori_loop(0, _KW, window, 0)
        plsc.subcore_barrier()
        pltpu.sync_copy(acc.at[pl.ds(s * _RPT, _RPT), :],
                        out_h.at[c, pl.ds(s * _RPT, _RPT), :])

    return conv(table, src2, dst2, a0, a1, gate)


def _bdot(a, b):
    return jnp.dot(a.astype(jnp.bfloat16), b.astype(jnp.bfloat16),
                   preferred_element_type=F32)


_NPK = _NPAD // 8
_RPK = _NPK // 16


def _prop1_tc(x128, w128, b128):
    def body(x_ref, w_ref, b_ref, o_ref):
        o_ref[...] = x_ref[...] * w_ref[...] + b_ref[...]
    return pl.pallas_call(
        body,
        grid=(_NPK // _RPK,),
        in_specs=[
            pl.BlockSpec((_RPK, 128), lambda i: (i, 0)),
            pl.BlockSpec((1, 128), lambda i: (0, 0)),
            pl.BlockSpec((1, 128), lambda i: (0, 0)),
        ],
        out_specs=pl.BlockSpec((_RPK, 128), lambda i: (i, 0)),
        out_shape=jax.ShapeDtypeStruct((_NPK, 128), F32),
    )(x128, w128, b128)


def _dense_tc(p, w1, b1, w2, b2):
    def body(p_ref, w1_ref, b1_ref, w2_ref, b2_ref, o_ref):
        agg = p_ref[0] + p_ref[1]
        h = jnp.maximum(_bdot(agg, w1_ref[...]) + b1_ref[...], 0.0)
        o_ref[...] = _bdot(h, w2_ref[...]) + b2_ref[...]
    return pl.pallas_call(
        body,
        grid=(_NPK // _RPK,),
        in_specs=[
            pl.BlockSpec((2, _RPK, 128), lambda i: (0, i, 0)),
            pl.BlockSpec((128, 128), lambda i: (0, 0)),
            pl.BlockSpec((1, 128), lambda i: (0, 0)),
            pl.BlockSpec((128, 128), lambda i: (0, 0)),
            pl.BlockSpec((1, 128), lambda i: (0, 0)),
        ],
        out_specs=pl.BlockSpec((_RPK, 128), lambda i: (i, 0)),
        out_shape=jax.ShapeDtypeStruct((_NPK, 128), F32),
    )(p, w1, b1, w2, b2)


def _dense3_tc(p, w1, b1):
    def body(p_ref, w1_ref, b1_ref, o_ref):
        agg = p_ref[0] + p_ref[1]
        o_ref[...] = jnp.maximum(_bdot(agg, w1_ref[...]) + b1_ref[...], 0.0)
    return pl.pallas_call(
        body,
        grid=(_NPK // _RPK,),
        in_specs=[
            pl.BlockSpec((2, _RPK, 128), lambda i: (0, i, 0)),
            pl.BlockSpec((128, 128), lambda i: (0, 0)),
            pl.BlockSpec((1, 128), lambda i: (0, 0)),
        ],
        out_specs=pl.BlockSpec((_RPK, 128), lambda i: (i, 0)),
        out_shape=jax.ShapeDtypeStruct((_NPK, 128), F32),
    )(p, w1, b1)


def _head_tc(h24, w1, b1, w2, b2):
    def body(x_ref, w1_ref, b1_ref, w2_ref, b2_ref, o_ref):
        h = jnp.maximum(_bdot(x_ref[...], w1_ref[...]) + b1_ref[...], 0.0)
        o = _bdot(h, w2_ref[...]) + b2_ref[...]
        m = jnp.max(o, axis=1, keepdims=True)
        lse = jnp.log(jnp.sum(jnp.exp(o - m), axis=1, keepdims=True)) + m
        o_ref[...] = o - lse
    rows = _N // 6
    return pl.pallas_call(
        body,
        out_shape=jax.ShapeDtypeStruct((rows, 4), F32),
    )(h24, w1, b1, w2, b2)


def _pad16(a, rows=16):
    out = jnp.zeros((rows, 16), F32)
    return out.at[:a.shape[0], :a.shape[1]].set(a)


def kernel(x, edge_index, edge_attr, Wx1, bx1, We1, be1, Wo1, bo1,
           Wx2, bx2, We2, be2, Wo2, bo2,
           Wx3, bx3, We3, be3, Wo3, bo3,
           Wl1, bl1, Wl2, bl2):
    src = edge_index[0]
    dst = edge_index[1]
    npad = _EPAD - _E
    fill_src = (jnp.arange(npad, dtype=jnp.int32) % _N)
    fill_dst = _N + (jnp.arange(npad, dtype=jnp.int32) % (_NPAD - _N))
    src_p = jnp.concatenate([src, fill_src]).reshape(_EPAD // _CH, _CH)
    dst_p = jnp.concatenate([dst, fill_dst]).reshape(_EPAD // _CH, _CH)
    zf = jnp.zeros((npad,), F32)
    def r16(v):
        return v.astype(jnp.bfloat16).astype(F32)

    a0 = jnp.concatenate([r16(edge_attr[:, 0]), zf])
    a1 = jnp.concatenate([r16(edge_attr[:, 1]), zf])

    def gate_mat(We, be):
        g = jnp.zeros((4, 16), F32)
        g = g.at[0, :9].set(r16(We[:, 0]))
        g = g.at[1, :9].set(r16(We[:, 1]))
        g = g.at[2, :9].set(be)
        return g

    ge1, ge2, ge3 = gate_mat(We1, be1), gate_mat(We2, be2), gate_mat(We3, be3)

    eye8 = jnp.eye(8, dtype=F32)

    def bd(w):
        return jnp.kron(eye8, w)

    def tile8(b):
        return jnp.tile(b, (1, 8))

    w1row = tile8(_pad16(Wx1.T, rows=1))
    b1row = tile8(_pad16(bx1[None, :], rows=1))
    Wo1e, bo1e = bd(_pad16(Wo1.T)), tile8(_pad16(bo1[None, :], rows=1))
    Wx2e, bx2e = bd(_pad16(Wx2.T)), tile8(_pad16(bx2[None, :], rows=1))
    Wo2e, bo2e = bd(_pad16(Wo2.T)), tile8(_pad16(bo2[None, :], rows=1))
    Wx3e, bx3e = bd(_pad16(Wx3.T)), tile8(_pad16(bx3[None, :], rows=1))
    Wo3e, bo3e = bd(_pad16(Wo3.T)), tile8(_pad16(bo3[None, :], rows=1))

    x8 = jnp.pad(x, ((0, _NPAD - _N), (0, 0))).reshape(_NPK, 8)
    x128 = jnp.repeat(x8, 16, axis=1)

    t1 = _prop1_tc(x128, w1row, b1row).reshape(_NPAD, 16)
    p1 = _sc_conv(t1, src_p, dst_p, a0, a1, ge1).reshape(2, _NPK, 128)
    t2 = _dense_tc(p1, Wo1e, bo1e, Wx2e, bx2e).reshape(_NPAD, 16)
    p2 = _sc_conv(t2, src_p, dst_p, a0, a1, ge2).reshape(2, _NPK, 128)
    t3 = _dense_tc(p2, Wo2e, bo2e, Wx3e, bx3e).reshape(_NPAD, 16)
    p3 = _sc_conv(t3, src_p, dst_p, a0, a1, ge3).reshape(2, _NPK, 128)
    h3 = _dense3_tc(p3, Wo3e, bo3e).reshape(_NPAD, 16)

    h24 = h3[:_N, :4].reshape(_N // 6, 24)
    return _head_tc(h24, Wl1.T, bl1[None, :], Wl2.T, bl2[None, :])


# speedup vs baseline: 25.4574x; 1.5186x over previous
import functools

import jax
import jax.numpy as jnp
from jax import lax
from jax.experimental import pallas as pl
from jax.experimental.pallas import tpu as pltpu
from jax.experimental.pallas import tpu_sc as plsc

F32 = jnp.float32

_N = 99996
_E = 6399744
_NPAD = 100352
_NW = 32
_W = 512
_KW = 392
_PW = _W * _KW
_EPAD = _NW * _PW
_CH = 128
_NC = _W // _CH
_RPT = 6272


def _sc_conv(table, src2, dst2, a0, a1, gate):
    mesh = plsc.VectorSubcoreMesh(core_axis_name="c", subcore_axis_name="s")

    @functools.partial(
        pl.kernel,
        mesh=mesh,
        compiler_params=pltpu.CompilerParams(use_tc_tiling_on_sc=False),
        out_type=jax.ShapeDtypeStruct((2, _NPAD, 16), F32),
        scratch_types=[
            pltpu.VMEM((2, _NC, _CH), jnp.int32),
            pltpu.VMEM((2, _NC, _CH), jnp.int32),
            pltpu.VMEM((2, _W), F32),
            pltpu.VMEM((2, _W), F32),
            pltpu.VMEM((2, _W, 16), F32),
            pltpu.VMEM((4, 16), F32),
            pltpu.VMEM_SHARED((_NPAD, 16), F32),
            pltpu.SemaphoreType.DMA,
            pltpu.SemaphoreType.DMA,
            pltpu.SemaphoreType.DMA,
            pltpu.SemaphoreType.DMA,
            pltpu.SemaphoreType.DMA,
        ],
    )
    def conv(table_h, src_h, dst_h, a0_h, a1_h, gate_h, out_h,
             src_v, dst_v, a0_v, a1_v, rows_v, g_v, acc,
             sem_g0, sem_g1, sem_i0, sem_i1, sem_s):
        c = lax.axis_index("c")
        s = lax.axis_index("s")
        w = c * 16 + s
        sem_g = (sem_g0, sem_g1)
        sem_i = (sem_i0, sem_i1)

        z0 = rows_v.at[0]

        def zbody(i, _):
            rows_v[0, i, :] = jnp.zeros((16,), F32)
            return 0
        lax.fori_loop(0, _W, zbody, 0)
        for t in range(12):
            pltpu.sync_copy(z0, acc.at[pl.ds(s * _RPT + t * _W, _W), :])
        pltpu.sync_copy(rows_v.at[0, pl.ds(0, 128), :],
                        acc.at[pl.ds(s * _RPT + 12 * _W, 128), :])
        plsc.subcore_barrier()

        pltpu.sync_copy(gate_h, g_v)
        u0 = g_v[0, :]
        u1 = g_v[1, :]
        cc = g_v[2, :]

        def in_copies(k, b):
            row0 = w * (_PW // _CH) + k * _NC
            e0 = w * _PW + k * _W
            sem = sem_i[b]
            return [
                pltpu.make_async_copy(src_h.at[pl.ds(row0, _NC), :],
                                      src_v.at[b], sem),
                pltpu.make_async_copy(dst_h.at[pl.ds(row0, _NC), :],
                                      dst_v.at[b], sem),
                pltpu.make_async_copy(a0_h.at[pl.ds(e0, _W)], a0_v.at[b], sem),
                pltpu.make_async_copy(a1_h.at[pl.ds(e0, _W)], a1_v.at[b], sem),
            ]

        def gathers(b):
            return [pltpu.make_async_copy(
                table_h.at[src_v.at[b, j]],
                rows_v.at[b, pl.ds(j * _CH, _CH), :], sem_g[b])
                for j in range(_NC)]

        def fire(ds):
            for d in ds:
                d.start()

        def drain(ds):
            for d in ds:
                d.wait()

        fire(in_copies(0, 0))
        drain(in_copies(0, 0))
        fire(gathers(0))
        fire(in_copies(1, 1))

        def do_window(k, b):
            nxt = jnp.minimum(k + 1, _KW - 1)
            nxt2 = jnp.minimum(k + 2, _KW - 1)
            drain(in_copies(nxt, 1 - b))
            fire(gathers(1 - b))
            drain(gathers(b))

            shs = []
            for j in range(_NC):
                def gbody(gg, _):
                    g = j * (_CH // 16) + gg
                    av0 = a0_v[b, pl.ds(g * 16, 16)]
                    av1 = a1_v[b, pl.ds(g * 16, 16)]
                    for l in range(16):
                        i = g * 16 + l
                        gv = u0 * av0[l] + u1 * av1[l] + cc
                        rows_v[b, i, :] = rows_v[b, i, :] * gv
                    return 0
                lax.fori_loop(0, _CH // 16, gbody, 0)
                d = pltpu.make_async_copy(
                    rows_v.at[b, pl.ds(j * _CH, _CH), :],
                    acc.at[dst_v.at[b, j]], sem_s)
                d.start(add=True)
                shs.append(d)
            drain(shs)
            fire(in_copies(nxt2, b))
            return 0

        def pair(kk, _):
            do_window(2 * kk, 0)
            do_window(2 * kk + 1, 1)
            return 0

        lax.fori_loop(0, _KW // 2, pair, 0)
        drain(gathers(0))
        drain(in_copies(_KW - 1, 1))

        plsc.subcore_barrier()
        pltpu.sync_copy(acc.at[pl.ds(s * _RPT, _RPT), :],
                        out_h.at[c, pl.ds(s * _RPT, _RPT), :])

    return conv(table, src2, dst2, a0, a1, gate)


def _bdot(a, b):
    return jnp.dot(a.astype(jnp.bfloat16), b.astype(jnp.bfloat16),
                   preferred_element_type=F32)


_NPK = _NPAD // 8
_RPK = _NPK // 16


def _prop1_tc(x128, w128, b128):
    def body(x_ref, w_ref, b_ref, o_ref):
        o_ref[...] = x_ref[...] * w_ref[...] + b_ref[...]
    return pl.pallas_call(
        body,
        grid=(_NPK // _RPK,),
        in_specs=[
            pl.BlockSpec((_RPK, 128), lambda i: (i, 0)),
            pl.BlockSpec((1, 128), lambda i: (0, 0)),
            pl.BlockSpec((1, 128), lambda i: (0, 0)),
        ],
        out_specs=pl.BlockSpec((_RPK, 128), lambda i: (i, 0)),
        out_shape=jax.ShapeDtypeStruct((_NPK, 128), F32),
    )(x128, w128, b128)


def _dense_tc(p, w1, b1, w2, b2):
    def body(p_ref, w1_ref, b1_ref, w2_ref, b2_ref, o_ref):
        agg = p_ref[0] + p_ref[1]
        h = jnp.maximum(_bdot(agg, w1_ref[...]) + b1_ref[...], 0.0)
        o_ref[...] = _bdot(h, w2_ref[...]) + b2_ref[...]
    return pl.pallas_call(
        body,
        grid=(_NPK // _RPK,),
        in_specs=[
            pl.BlockSpec((2, _RPK, 128), lambda i: (0, i, 0)),
            pl.BlockSpec((128, 128), lambda i: (0, 0)),
            pl.BlockSpec((1, 128), lambda i: (0, 0)),
            pl.BlockSpec((128, 128), lambda i: (0, 0)),
            pl.BlockSpec((1, 128), lambda i: (0, 0)),
        ],
        out_specs=pl.BlockSpec((_RPK, 128), lambda i: (i, 0)),
        out_shape=jax.ShapeDtypeStruct((_NPK, 128), F32),
    )(p, w1, b1, w2, b2)


def _dense3_tc(p, w1, b1):
    def body(p_ref, w1_ref, b1_ref, o_ref):
        agg = p_ref[0] + p_ref[1]
        o_ref[...] = jnp.maximum(_bdot(agg, w1_ref[...]) + b1_ref[...], 0.0)
    return pl.pallas_call(
        body,
        grid=(_NPK // _RPK,),
        in_specs=[
            pl.BlockSpec((2, _RPK, 128), lambda i: (0, i, 0)),
            pl.BlockSpec((128, 128), lambda i: (0, 0)),
            pl.BlockSpec((1, 128), lambda i: (0, 0)),
        ],
        out_specs=pl.BlockSpec((_RPK, 128), lambda i: (i, 0)),
        out_shape=jax.ShapeDtypeStruct((_NPK, 128), F32),
    )(p, w1, b1)


def _head_tc(h24, w1, b1, w2, b2):
    def body(x_ref, w1_ref, b1_ref, w2_ref, b2_ref, o_ref):
        h = jnp.maximum(_bdot(x_ref[...], w1_ref[...]) + b1_ref[...], 0.0)
        o = _bdot(h, w2_ref[...]) + b2_ref[...]
        m = jnp.max(o, axis=1, keepdims=True)
        lse = jnp.log(jnp.sum(jnp.exp(o - m), axis=1, keepdims=True)) + m
        o_ref[...] = o - lse
    rows = _N // 6
    return pl.pallas_call(
        body,
        out_shape=jax.ShapeDtypeStruct((rows, 4), F32),
    )(h24, w1, b1, w2, b2)


def _pad16(a, rows=16):
    out = jnp.zeros((rows, 16), F32)
    return out.at[:a.shape[0], :a.shape[1]].set(a)


def kernel(x, edge_index, edge_attr, Wx1, bx1, We1, be1, Wo1, bo1,
           Wx2, bx2, We2, be2, Wo2, bo2,
           Wx3, bx3, We3, be3, Wo3, bo3,
           Wl1, bl1, Wl2, bl2):
    src = edge_index[0]
    dst = edge_index[1]
    npad = _EPAD - _E
    fill_src = (jnp.arange(npad, dtype=jnp.int32) % _N)
    fill_dst = _N + (jnp.arange(npad, dtype=jnp.int32) % (_NPAD - _N))
    src_p = jnp.concatenate([src, fill_src]).reshape(_EPAD // _CH, _CH)
    dst_p = jnp.concatenate([dst, fill_dst]).reshape(_EPAD // _CH, _CH)
    zf = jnp.zeros((npad,), F32)
    def r16(v):
        return v.astype(jnp.bfloat16).astype(F32)

    a0 = jnp.concatenate([r16(edge_attr[:, 0]), zf])
    a1 = jnp.concatenate([r16(edge_attr[:, 1]), zf])

    def gate_mat(We, be):
        g = jnp.zeros((4, 16), F32)
        g = g.at[0, :9].set(r16(We[:, 0]))
        g = g.at[1, :9].set(r16(We[:, 1]))
        g = g.at[2, :9].set(be)
        return g

    ge1, ge2, ge3 = gate_mat(We1, be1), gate_mat(We2, be2), gate_mat(We3, be3)

    eye8 = jnp.eye(8, dtype=F32)

    def bd(w):
        return jnp.kron(eye8, w)

    def tile8(b):
        return jnp.tile(b, (1, 8))

    w1row = tile8(_pad16(Wx1.T, rows=1))
    b1row = tile8(_pad16(bx1[None, :], rows=1))
    Wo1e, bo1e = bd(_pad16(Wo1.T)), tile8(_pad16(bo1[None, :], rows=1))
    Wx2e, bx2e = bd(_pad16(Wx2.T)), tile8(_pad16(bx2[None, :], rows=1))
    Wo2e, bo2e = bd(_pad16(Wo2.T)), tile8(_pad16(bo2[None, :], rows=1))
    Wx3e, bx3e = bd(_pad16(Wx3.T)), tile8(_pad16(bx3[None, :], rows=1))
    Wo3e, bo3e = bd(_pad16(Wo3.T)), tile8(_pad16(bo3[None, :], rows=1))

    x8 = jnp.pad(x, ((0, _NPAD - _N), (0, 0))).reshape(_NPK, 8)
    x128 = jnp.repeat(x8, 16, axis=1)

    t1 = _prop1_tc(x128, w1row, b1row).reshape(_NPAD, 16)
    p1 = _sc_conv(t1, src_p, dst_p, a0, a1, ge1).reshape(2, _NPK, 128)
    t2 = _dense_tc(p1, Wo1e, bo1e, Wx2e, bx2e).reshape(_NPAD, 16)
    p2 = _sc_conv(t2, src_p, dst_p, a0, a1, ge2).reshape(2, _NPK, 128)
    t3 = _dense_tc(p2, Wo2e, bo2e, Wx3e, bx3e).reshape(_NPAD, 16)
    p3 = _sc_conv(t3, src_p, dst_p, a0, a1, ge3).reshape(2, _NPK, 128)
    h3 = _dense3_tc(p3, Wo3e, bo3e).reshape(_NPAD, 16)

    h24 = h3[:_N, :4].reshape(_N // 6, 24)
    return _head_tc(h24, Wl1.T, bl1[None, :], Wl2.T, bl2[None, :])
